# Initial kernel scaffold; baseline (speedup 1.0000x reference)
import jax
import jax.numpy as jnp
from jax.experimental import pallas as pl


def kernel(x, edge_index, edge_attr, Wp, bp, Wl1, bl1, Wr1, br1, We1, att1, bo1, Wl2, bl2, Wr2, br2, We2, att2, bo2, W1, b1, W2, b2, W3, b3):
    raise NotImplementedError("write your pallas kernel here")


# speedup vs baseline: 28.4375x; 28.4375x over previous
import functools

import jax
import jax.numpy as jnp
import numpy as np
from jax import lax
from jax.experimental import pallas as pl
from jax.experimental.pallas import tpu as pltpu
from jax.experimental.pallas import tpu_sc as plsc

_N = 10000
_E = 320000
_HID = 128
_H = 8
_C = 16
_NC = 2
_NS = 16
_NW = _NC * _NS
_B = 40
_NB = _E // _B
_NITER = _NB // _NW
_B2 = 80
_NB2 = _E // _B2
_NITER2 = _NB2 // _NW
_ROWS = 632
_ROWS_LAST = _N - 15 * _ROWS

_BN = 1000


def _elu(v):
    return jnp.where(v > 0, v, jnp.exp(v) - 1.0)


def _bcast_lane(v, lane):
    idx = jnp.full((16,), lane, dtype=jnp.int32)
    return lax.gather(
        v, idx[:, None],
        dimension_numbers=lax.GatherDimensionNumbers(
            offset_dims=(), collapsed_slice_dims=(0,), start_index_map=(0,)),
        slice_sizes=(1,),
        mode=lax.GatherScatterMode.PROMISE_IN_BOUNDS)


def _proj_body(x_ref, Wp_ref, bp_ref, Wl_ref, bl_ref, Wr_ref, br_ref,
               xl_ref, xr_ref):
    h = jnp.dot(x_ref[...], Wp_ref[...], preferred_element_type=jnp.float32)
    h = _elu(h + bp_ref[...])
    xl_ref[...] = jnp.dot(h, Wl_ref[...],
                          preferred_element_type=jnp.float32) + bl_ref[...]
    xr_ref[...] = jnp.dot(h, Wr_ref[...],
                          preferred_element_type=jnp.float32) + br_ref[...]


def _tc_proj(x, Wp, bp, Wl, bl, Wr, br):
    full = lambda r, c: pl.BlockSpec((r, c), lambda i: (0, 0))
    return pl.pallas_call(
        _proj_body,
        grid=(_N // _BN,),
        in_specs=[
            pl.BlockSpec((_BN, _HID), lambda i: (i, 0)),
            full(_HID, _HID), full(1, _HID),
            full(_HID, _HID), full(1, _HID),
            full(_HID, _HID), full(1, _HID),
        ],
        out_specs=[pl.BlockSpec((_BN, _HID), lambda i: (i, 0))] * 2,
        out_shape=[jax.ShapeDtypeStruct((_N, _HID), jnp.float32)] * 2,
    )(x, Wp, bp.reshape(1, _HID), Wl, bl.reshape(1, _HID),
      Wr, br.reshape(1, _HID))


def _combine_h(sd_ref, dd_ref, S_ref, bo_ref):
    acc = sd_ref[0] + sd_ref[1]
    den = dd_ref[0] + dd_ref[1]
    denf = jnp.dot(den, S_ref[...], preferred_element_type=jnp.float32)
    return _elu(acc / (denf + 1e-16) + bo_ref[...])


def _comb_body(sd_ref, dd_ref, S_ref, bo_ref, Wl_ref, bl_ref, Wr_ref,
               br_ref, xl_ref, xr_ref):
    h = _combine_h(sd_ref, dd_ref, S_ref, bo_ref)
    xl_ref[...] = jnp.dot(h, Wl_ref[...],
                          preferred_element_type=jnp.float32) + bl_ref[...]
    xr_ref[...] = jnp.dot(h, Wr_ref[...],
                          preferred_element_type=jnp.float32) + br_ref[...]


def _tc_combine(sd, dd, S, bo, Wl, bl, Wr, br):
    full = lambda r, c: pl.BlockSpec((r, c), lambda i: (0, 0))
    return pl.pallas_call(
        _comb_body,
        grid=(_N // _BN,),
        in_specs=[
            pl.BlockSpec((_NC, _BN, _HID), lambda i: (0, i, 0)),
            pl.BlockSpec((_NC, _BN, 16), lambda i: (0, i, 0)),
            full(16, _HID), full(1, _HID),
            full(_HID, _HID), full(1, _HID),
            full(_HID, _HID), full(1, _HID),
        ],
        out_specs=[pl.BlockSpec((_BN, _HID), lambda i: (i, 0))] * 2,
        out_shape=[jax.ShapeDtypeStruct((_N, _HID), jnp.float32)] * 2,
    )(sd, dd, S, bo.reshape(1, _HID), Wl, bl.reshape(1, _HID),
      Wr, br.reshape(1, _HID))


def _final_body(sd_ref, dd_ref, S_ref, bo_ref, W1_ref, b1_ref, W2_ref,
                b2_ref, W3_ref, b3_ref, out_ref):
    h = _combine_h(sd_ref, dd_ref, S_ref, bo_ref)
    a = jnp.maximum(jnp.dot(h, W1_ref[...],
                            preferred_element_type=jnp.float32)
                    + b1_ref[...], 0.0)
    a = jnp.maximum(jnp.dot(a, W2_ref[...],
                            preferred_element_type=jnp.float32)
                    + b2_ref[...], 0.0)
    s = jnp.dot(a, W3_ref[...],
                preferred_element_type=jnp.float32) + b3_ref[...]
    out_ref[...] = jax.nn.sigmoid(s)


def _tc_final(sd, dd, S, bo, W1, b1, W2, b2, W3, b3):
    full = lambda r, c: pl.BlockSpec((r, c), lambda i: (0, 0))
    return pl.pallas_call(
        _final_body,
        grid=(_N // _BN,),
        in_specs=[
            pl.BlockSpec((_NC, _BN, _HID), lambda i: (0, i, 0)),
            pl.BlockSpec((_NC, _BN, 16), lambda i: (0, i, 0)),
            full(16, _HID), full(1, _HID),
            full(_HID, 512), full(1, 512),
            full(512, _HID), full(1, _HID),
            full(_HID, 1), full(1, 1),
        ],
        out_specs=pl.BlockSpec((_BN, 1), lambda i: (i, 0)),
        out_shape=jax.ShapeDtypeStruct((_N, 1), jnp.float32),
    )(sd, dd, S, bo.reshape(1, _HID), W1, b1.reshape(1, 512),
      W2, b2.reshape(1, _HID), W3, b3.reshape(1, 1))


def _sc_msg_body(xl_hbm, xr_hbm, src_hbm, dst_hbm, ea_hbm, we_hbm, att_hbm,
                 sd_out, ev_out,
                 sh, srcb, dstb, xlb, xrb, eab, wrowb, denb, web, attb,
                 sem, sem2):
    cid = lax.axis_index("c")
    sid = lax.axis_index("s")
    wid = sid * _NC + cid

    zero = jnp.zeros((16,), jnp.float32)
    lane = lax.iota(jnp.int32, 16)

    pltpu.sync_copy(we_hbm, web)
    pltpu.sync_copy(att_hbm, attb)

    def zrow(i, _):
        for j in range(_HID // 16):
            wrowb[i, j * 16:(j + 1) * 16] = zero
        return 0
    lax.fori_loop(0, _B, zrow, 0)

    rows = jnp.where(sid < 15, _ROWS, _ROWS_LAST)
    rbase = sid * _ROWS

    def zchunk(i, _):
        @pl.when(i * 8 < rows)
        def _():
            pltpu.sync_copy(wrowb.at[pl.ds(0, 8)],
                            sh.at[pl.ds(rbase + i * 8, 8)])
        return 0
    lax.fori_loop(0, _ROWS // 8, zchunk, 0)

    plsc.subcore_barrier()

    wevs = [[web[k, h * 16:(h + 1) * 16] for h in range(_H)] for k in range(4)]
    attvs = [attb[h, :] for h in range(_H)]

    def batch_body(i, _):
        b = wid + i * _NW
        base = b * _B
        pltpu.sync_copy(src_hbm.at[pl.ds(base, _B)], srcb)
        pltpu.sync_copy(dst_hbm.at[pl.ds(base, _B)], dstb)
        cp1 = pltpu.async_copy(xl_hbm.at[srcb], xlb, sem)
        cp2 = pltpu.async_copy(xr_hbm.at[dstb], xrb, sem2)
        pltpu.sync_copy(ea_hbm.at[pl.ds(base, _B)], eab)
        cp1.wait()
        cp2.wait()

        def edge_body(e, _):
            eav = eab[e, :]
            bc = [_bcast_lane(eav, k) for k in range(4)]
            denv = zero
            for h in range(_H):
                xlv = xlb[e, h * 16:(h + 1) * 16]
                xrv = xrb[e, h * 16:(h + 1) * 16]
                eev = (bc[0] * wevs[0][h] + bc[1] * wevs[1][h]
                       + bc[2] * wevs[2][h] + bc[3] * wevs[3][h])
                mv = xlv + xrv + eev
                mv = jnp.maximum(mv, 0.2 * mv)
                sv = plsc.cumsum(mv * attvs[h])
                ebv = jnp.exp(_bcast_lane(sv, 15))
                wrowb[e, h * 16:(h + 1) * 16] = ebv * xlv
                denv = jnp.where(lane == h, ebv, denv)
            denb[e, :] = denv
            return 0
        lax.fori_loop(0, _B, edge_body, 0)

        pltpu.sync_copy(wrowb, sh.at[dstb], add=True)
        pltpu.sync_copy(denb, ev_out.at[pl.ds(base, _B)])
        return 0
    lax.fori_loop(0, _NITER, batch_body, 0)

    plsc.subcore_barrier()

    @pl.when(sid < 15)
    def _():
        pltpu.sync_copy(sh.at[pl.ds(rbase, _ROWS)],
                        sd_out.at[cid, pl.ds(rbase, _ROWS)])

    @pl.when(sid == 15)
    def _():
        pltpu.sync_copy(sh.at[pl.ds(15 * _ROWS, _ROWS_LAST)],
                        sd_out.at[cid, pl.ds(15 * _ROWS, _ROWS_LAST)])


def _sc_msg_pass(xl, xr, src, dst, eap, We, att):
    mesh = plsc.VectorSubcoreMesh(core_axis_name="c", subcore_axis_name="s",
                                  num_cores=_NC, num_subcores=_NS)
    k = pl.kernel(
        _sc_msg_body,
        out_type=(jax.ShapeDtypeStruct((_NC, _N, _HID), jnp.float32),
                  jax.ShapeDtypeStruct((_E, 16), jnp.float32)),
        mesh=mesh,
        scratch_types=[
            pltpu.VMEM_SHARED((_N, _HID), jnp.float32),
            pltpu.VMEM((_B,), jnp.int32),
            pltpu.VMEM((_B,), jnp.int32),
            pltpu.VMEM((_B, _HID), jnp.float32),
            pltpu.VMEM((_B, _HID), jnp.float32),
            pltpu.VMEM((_B, 16), jnp.float32),
            pltpu.VMEM((_B, _HID), jnp.float32),
            pltpu.VMEM((_B, 16), jnp.float32),
            pltpu.VMEM((4, _HID), jnp.float32),
            pltpu.VMEM((_H, _C), jnp.float32),
            pltpu.SemaphoreType.DMA,
            pltpu.SemaphoreType.DMA,
        ],
        compiler_params=pltpu.CompilerParams(needs_layout_passes=False),
    )
    return k(xl, xr, src, dst, eap, We, att)


def _sc_den_body(ev_hbm, dst_hbm, dd_out,
                 sh, dstb, evb, sem):
    cid = lax.axis_index("c")
    sid = lax.axis_index("s")
    wid = sid * _NC + cid

    zero = jnp.zeros((16,), jnp.float32)

    def zrow(i, _):
        evb[i, :] = zero
        return 0
    lax.fori_loop(0, 8, zrow, 0)

    rows = jnp.where(sid < 15, _ROWS, _ROWS_LAST)
    rbase = sid * _ROWS

    def zchunk(i, _):
        @pl.when(i * 8 < rows)
        def _():
            pltpu.sync_copy(evb.at[pl.ds(0, 8)],
                            sh.at[pl.ds(rbase + i * 8, 8)])
        return 0
    lax.fori_loop(0, _ROWS // 8, zchunk, 0)

    plsc.subcore_barrier()

    def batch_body(i, _):
        b = wid + i * _NW
        base = b * _B2
        pltpu.sync_copy(dst_hbm.at[pl.ds(base, _B2)], dstb)
        pltpu.sync_copy(ev_hbm.at[pl.ds(base, _B2)], evb)
        pltpu.sync_copy(evb, sh.at[dstb], add=True)
        return 0
    lax.fori_loop(0, _NITER2, batch_body, 0)

    plsc.subcore_barrier()

    @pl.when(sid < 15)
    def _():
        pltpu.sync_copy(sh.at[pl.ds(rbase, _ROWS)],
                        dd_out.at[cid, pl.ds(rbase, _ROWS)])

    @pl.when(sid == 15)
    def _():
        pltpu.sync_copy(sh.at[pl.ds(15 * _ROWS, _ROWS_LAST)],
                        dd_out.at[cid, pl.ds(15 * _ROWS, _ROWS_LAST)])


def _sc_den_pass(ev, dst):
    mesh = plsc.VectorSubcoreMesh(core_axis_name="c", subcore_axis_name="s",
                                  num_cores=_NC, num_subcores=_NS)
    k = pl.kernel(
        _sc_den_body,
        out_type=jax.ShapeDtypeStruct((_NC, _N, 16), jnp.float32),
        mesh=mesh,
        scratch_types=[
            pltpu.VMEM_SHARED((_N, 16), jnp.float32),
            pltpu.VMEM((_B2,), jnp.int32),
            pltpu.VMEM((_B2, 16), jnp.float32),
            pltpu.SemaphoreType.DMA,
        ],
        compiler_params=pltpu.CompilerParams(needs_layout_passes=False),
    )
    return k(ev, dst)


def kernel(x, edge_index, edge_attr, Wp, bp,
           Wl1, bl1, Wr1, br1, We1, att1, bo1,
           Wl2, bl2, Wr2, br2, We2, att2, bo2,
           W1, b1, W2, b2, W3, b3):
    src = edge_index[0].astype(jnp.int32)
    dst = edge_index[1].astype(jnp.int32)
    eap = jnp.pad(edge_attr, ((0, 0), (0, 16 - edge_attr.shape[1])))

    S = np.zeros((16, _HID), np.float32)
    for h in range(_H):
        S[h, h * 16:(h + 1) * 16] = 1.0
    S = jnp.asarray(S)

    xl1, xr1 = _tc_proj(x, Wp, bp, Wl1, bl1, Wr1, br1)
    sd1, ev1 = _sc_msg_pass(xl1, xr1, src, dst, eap, We1, att1)
    dd1 = _sc_den_pass(ev1, dst)
    xl2, xr2 = _tc_combine(sd1, dd1, S, bo1, Wl2, bl2, Wr2, br2)
    sd2, ev2 = _sc_msg_pass(xl2, xr2, src, dst, eap, We2, att2)
    dd2 = _sc_den_pass(ev2, dst)
    return _tc_final(sd2, dd2, S, bo2, W1, b1, W2, b2, W3, b3)


# speedup vs baseline: 30.9988x; 1.0901x over previous
import functools

import jax
import jax.numpy as jnp
import numpy as np
from jax import lax
from jax.experimental import pallas as pl
from jax.experimental.pallas import tpu as pltpu
from jax.experimental.pallas import tpu_sc as plsc

_N = 10000
_E = 320000
_HID = 128
_H = 8
_C = 16
_NC = 2
_NS = 16
_NW = _NC * _NS
_B = 40
_NB = _E // _B
_NITER = _NB // _NW
_B2 = 80
_NB2 = _E // _B2
_NITER2 = _NB2 // _NW
_ROWS = 632
_ROWS_LAST = _N - 15 * _ROWS

_BN = 1000


def _elu(v):
    return jnp.where(v > 0, v, jnp.exp(v) - 1.0)


def _bcast_lane(v, lane):
    idx = jnp.full((16,), lane, dtype=jnp.int32)
    return lax.gather(
        v, idx[:, None],
        dimension_numbers=lax.GatherDimensionNumbers(
            offset_dims=(), collapsed_slice_dims=(0,), start_index_map=(0,)),
        slice_sizes=(1,),
        mode=lax.GatherScatterMode.PROMISE_IN_BOUNDS)


def _proj_body(x_ref, Wp_ref, bp_ref, Wl_ref, bl_ref, Wr_ref, br_ref,
               xl_ref, xr_ref):
    h = jnp.dot(x_ref[...], Wp_ref[...], preferred_element_type=jnp.float32)
    h = _elu(h + bp_ref[...])
    xl_ref[...] = jnp.dot(h, Wl_ref[...],
                          preferred_element_type=jnp.float32) + bl_ref[...]
    xr_ref[...] = jnp.dot(h, Wr_ref[...],
                          preferred_element_type=jnp.float32) + br_ref[...]


def _tc_proj(x, Wp, bp, Wl, bl, Wr, br):
    full = lambda r, c: pl.BlockSpec((r, c), lambda i: (0, 0))
    return pl.pallas_call(
        _proj_body,
        grid=(_N // _BN,),
        in_specs=[
            pl.BlockSpec((_BN, _HID), lambda i: (i, 0)),
            full(_HID, _HID), full(1, _HID),
            full(_HID, _HID), full(1, _HID),
            full(_HID, _HID), full(1, _HID),
        ],
        out_specs=[pl.BlockSpec((_BN, _HID), lambda i: (i, 0))] * 2,
        out_shape=[jax.ShapeDtypeStruct((_N, _HID), jnp.float32)] * 2,
    )(x, Wp, bp.reshape(1, _HID), Wl, bl.reshape(1, _HID),
      Wr, br.reshape(1, _HID))


def _combine_h(sd_ref, dd_ref, S_ref, bo_ref):
    acc = sd_ref[0] + sd_ref[1]
    den = dd_ref[0] + dd_ref[1]
    denf = jnp.dot(den, S_ref[...], preferred_element_type=jnp.float32)
    return _elu(acc / (denf + 1e-16) + bo_ref[...])


def _comb_body(sd_ref, dd_ref, S_ref, bo_ref, Wl_ref, bl_ref, Wr_ref,
               br_ref, xl_ref, xr_ref):
    h = _combine_h(sd_ref, dd_ref, S_ref, bo_ref)
    xl_ref[...] = jnp.dot(h, Wl_ref[...],
                          preferred_element_type=jnp.float32) + bl_ref[...]
    xr_ref[...] = jnp.dot(h, Wr_ref[...],
                          preferred_element_type=jnp.float32) + br_ref[...]


def _tc_combine(sd, dd, S, bo, Wl, bl, Wr, br):
    full = lambda r, c: pl.BlockSpec((r, c), lambda i: (0, 0))
    return pl.pallas_call(
        _comb_body,
        grid=(_N // _BN,),
        in_specs=[
            pl.BlockSpec((_NC, _BN, _HID), lambda i: (0, i, 0)),
            pl.BlockSpec((_NC, _BN, 16), lambda i: (0, i, 0)),
            full(16, _HID), full(1, _HID),
            full(_HID, _HID), full(1, _HID),
            full(_HID, _HID), full(1, _HID),
        ],
        out_specs=[pl.BlockSpec((_BN, _HID), lambda i: (i, 0))] * 2,
        out_shape=[jax.ShapeDtypeStruct((_N, _HID), jnp.float32)] * 2,
    )(sd, dd, S, bo.reshape(1, _HID), Wl, bl.reshape(1, _HID),
      Wr, br.reshape(1, _HID))


def _final_body(sd_ref, dd_ref, S_ref, bo_ref, W1_ref, b1_ref, W2_ref,
                b2_ref, W3_ref, b3_ref, out_ref):
    h = _combine_h(sd_ref, dd_ref, S_ref, bo_ref)
    a = jnp.maximum(jnp.dot(h, W1_ref[...],
                            preferred_element_type=jnp.float32)
                    + b1_ref[...], 0.0)
    a = jnp.maximum(jnp.dot(a, W2_ref[...],
                            preferred_element_type=jnp.float32)
                    + b2_ref[...], 0.0)
    s = jnp.dot(a, W3_ref[...],
                preferred_element_type=jnp.float32) + b3_ref[...]
    out_ref[...] = jax.nn.sigmoid(s)


def _tc_final(sd, dd, S, bo, W1, b1, W2, b2, W3, b3):
    full = lambda r, c: pl.BlockSpec((r, c), lambda i: (0, 0))
    return pl.pallas_call(
        _final_body,
        grid=(_N // _BN,),
        in_specs=[
            pl.BlockSpec((_NC, _BN, _HID), lambda i: (0, i, 0)),
            pl.BlockSpec((_NC, _BN, 16), lambda i: (0, i, 0)),
            full(16, _HID), full(1, _HID),
            full(_HID, 512), full(1, 512),
            full(512, _HID), full(1, _HID),
            full(_HID, 1), full(1, 1),
        ],
        out_specs=pl.BlockSpec((_BN, 1), lambda i: (i, 0)),
        out_shape=jax.ShapeDtypeStruct((_N, 1), jnp.float32),
    )(sd, dd, S, bo.reshape(1, _HID), W1, b1.reshape(1, 512),
      W2, b2.reshape(1, _HID), W3, b3.reshape(1, 1))


def _sc_msg_body(xl_hbm, xr_hbm, src_hbm, dst_hbm, ea_hbm, we_hbm, att_hbm,
                 sd_out, ev_out,
                 sh, srcb0, dstb0, xlb0, xrb0, eab0,
                 srcb1, dstb1, xlb1, xrb1, eab1,
                 wrowb, denb, web, attb,
                 seml0, semr0, seml1, semr1):
    cid = lax.axis_index("c")
    sid = lax.axis_index("s")
    wid = sid * _NC + cid

    zero = jnp.zeros((16,), jnp.float32)
    lane = lax.iota(jnp.int32, 16)

    bufs = [(srcb0, dstb0, xlb0, xrb0, eab0, seml0, semr0),
            (srcb1, dstb1, xlb1, xrb1, eab1, seml1, semr1)]

    pltpu.sync_copy(we_hbm, web)
    pltpu.sync_copy(att_hbm, attb)

    def zrow(i, _):
        for j in range(_HID // 16):
            wrowb[i, j * 16:(j + 1) * 16] = zero
        return 0
    lax.fori_loop(0, _B, zrow, 0)

    rows = jnp.where(sid < 15, _ROWS, _ROWS_LAST)
    rbase = sid * _ROWS

    def zchunk(i, _):
        @pl.when(i * 8 < rows)
        def _():
            pltpu.sync_copy(wrowb.at[pl.ds(0, 8)],
                            sh.at[pl.ds(rbase + i * 8, 8)])
        return 0
    lax.fori_loop(0, _ROWS // 8, zchunk, 0)

    plsc.subcore_barrier()

    wevs = [[web[k, h * 16:(h + 1) * 16] for h in range(_H)] for k in range(4)]
    attvs = [attb[h, :] for h in range(_H)]

    def load(i, p):
        srcb, dstb, xlb, xrb, eab, seml, semr = bufs[p]
        b = wid + jnp.remainder(i, _NITER) * _NW
        base = b * _B
        pltpu.sync_copy(src_hbm.at[pl.ds(base, _B)], srcb)
        pltpu.sync_copy(dst_hbm.at[pl.ds(base, _B)], dstb)
        pltpu.async_copy(xl_hbm.at[srcb], xlb, seml)
        pltpu.async_copy(xr_hbm.at[dstb], xrb, semr)
        pltpu.sync_copy(ea_hbm.at[pl.ds(base, _B)], eab)

    def process(i, p):
        srcb, dstb, xlb, xrb, eab, seml, semr = bufs[p]
        pltpu.make_async_copy(xl_hbm.at[srcb], xlb, seml).wait()
        pltpu.make_async_copy(xr_hbm.at[dstb], xrb, semr).wait()
        base = (wid + i * _NW) * _B

        def edge_body(e, _):
            eav = eab[e, :]
            bc = [_bcast_lane(eav, k) for k in range(4)]
            denv = zero
            for h in range(_H):
                xlv = xlb[e, h * 16:(h + 1) * 16]
                xrv = xrb[e, h * 16:(h + 1) * 16]
                eev = (bc[0] * wevs[0][h] + bc[1] * wevs[1][h]
                       + bc[2] * wevs[2][h] + bc[3] * wevs[3][h])
                mv = xlv + xrv + eev
                mv = jnp.maximum(mv, 0.2 * mv)
                sv = plsc.cumsum(mv * attvs[h])
                ebv = jnp.exp(_bcast_lane(sv, 15))
                wrowb[e, h * 16:(h + 1) * 16] = ebv * xlv
                denv = jnp.where(lane == h, ebv, denv)
            denb[e, :] = denv
            return 0
        lax.fori_loop(0, _B, edge_body, 0)

        pltpu.sync_copy(wrowb, sh.at[dstb], add=True)
        pltpu.sync_copy(denb, ev_out.at[pl.ds(base, _B)])

    load(0, 0)

    def batch_body(j, _):
        load(2 * j + 1, 1)
        process(2 * j, 0)
        load(2 * j + 2, 0)
        process(2 * j + 1, 1)
        return 0
    lax.fori_loop(0, _NITER // 2, batch_body, 0)

    pltpu.make_async_copy(xl_hbm.at[srcb0], xlb0, seml0).wait()
    pltpu.make_async_copy(xr_hbm.at[dstb0], xrb0, semr0).wait()

    plsc.subcore_barrier()

    @pl.when(sid < 15)
    def _():
        pltpu.sync_copy(sh.at[pl.ds(rbase, _ROWS)],
                        sd_out.at[cid, pl.ds(rbase, _ROWS)])

    @pl.when(sid == 15)
    def _():
        pltpu.sync_copy(sh.at[pl.ds(15 * _ROWS, _ROWS_LAST)],
                        sd_out.at[cid, pl.ds(15 * _ROWS, _ROWS_LAST)])


def _sc_msg_pass(xl, xr, src, dst, eap, We, att):
    mesh = plsc.VectorSubcoreMesh(core_axis_name="c", subcore_axis_name="s",
                                  num_cores=_NC, num_subcores=_NS)
    k = pl.kernel(
        _sc_msg_body,
        out_type=(jax.ShapeDtypeStruct((_NC, _N, _HID), jnp.float32),
                  jax.ShapeDtypeStruct((_E, 16), jnp.float32)),
        mesh=mesh,
        scratch_types=[
            pltpu.VMEM_SHARED((_N, _HID), jnp.float32),
            pltpu.VMEM((_B,), jnp.int32),
            pltpu.VMEM((_B,), jnp.int32),
            pltpu.VMEM((_B, _HID), jnp.float32),
            pltpu.VMEM((_B, _HID), jnp.float32),
            pltpu.VMEM((_B, 16), jnp.float32),
            pltpu.VMEM((_B,), jnp.int32),
            pltpu.VMEM((_B,), jnp.int32),
            pltpu.VMEM((_B, _HID), jnp.float32),
            pltpu.VMEM((_B, _HID), jnp.float32),
            pltpu.VMEM((_B, 16), jnp.float32),
            pltpu.VMEM((_B, _HID), jnp.float32),
            pltpu.VMEM((_B, 16), jnp.float32),
            pltpu.VMEM((4, _HID), jnp.float32),
            pltpu.VMEM((_H, _C), jnp.float32),
            pltpu.SemaphoreType.DMA,
            pltpu.SemaphoreType.DMA,
            pltpu.SemaphoreType.DMA,
            pltpu.SemaphoreType.DMA,
        ],
        compiler_params=pltpu.CompilerParams(needs_layout_passes=False),
    )
    return k(xl, xr, src, dst, eap, We, att)


def _sc_den_body(ev_hbm, dst_hbm, dd_out,
                 sh, dstb, evb, sem):
    cid = lax.axis_index("c")
    sid = lax.axis_index("s")
    wid = sid * _NC + cid

    zero = jnp.zeros((16,), jnp.float32)

    def zrow(i, _):
        evb[i, :] = zero
        return 0
    lax.fori_loop(0, 8, zrow, 0)

    rows = jnp.where(sid < 15, _ROWS, _ROWS_LAST)
    rbase = sid * _ROWS

    def zchunk(i, _):
        @pl.when(i * 8 < rows)
        def _():
            pltpu.sync_copy(evb.at[pl.ds(0, 8)],
                            sh.at[pl.ds(rbase + i * 8, 8)])
        return 0
    lax.fori_loop(0, _ROWS // 8, zchunk, 0)

    plsc.subcore_barrier()

    def batch_body(i, _):
        b = wid + i * _NW
        base = b * _B2
        pltpu.sync_copy(dst_hbm.at[pl.ds(base, _B2)], dstb)
        pltpu.sync_copy(ev_hbm.at[pl.ds(base, _B2)], evb)
        pltpu.sync_copy(evb, sh.at[dstb], add=True)
        return 0
    lax.fori_loop(0, _NITER2, batch_body, 0)

    plsc.subcore_barrier()

    @pl.when(sid < 15)
    def _():
        pltpu.sync_copy(sh.at[pl.ds(rbase, _ROWS)],
                        dd_out.at[cid, pl.ds(rbase, _ROWS)])

    @pl.when(sid == 15)
    def _():
        pltpu.sync_copy(sh.at[pl.ds(15 * _ROWS, _ROWS_LAST)],
                        dd_out.at[cid, pl.ds(15 * _ROWS, _ROWS_LAST)])


def _sc_den_pass(ev, dst):
    mesh = plsc.VectorSubcoreMesh(core_axis_name="c", subcore_axis_name="s",
                                  num_cores=_NC, num_subcores=_NS)
    k = pl.kernel(
        _sc_den_body,
        out_type=jax.ShapeDtypeStruct((_NC, _N, 16), jnp.float32),
        mesh=mesh,
        scratch_types=[
            pltpu.VMEM_SHARED((_N, 16), jnp.float32),
            pltpu.VMEM((_B2,), jnp.int32),
            pltpu.VMEM((_B2, 16), jnp.float32),
            pltpu.SemaphoreType.DMA,
        ],
        compiler_params=pltpu.CompilerParams(needs_layout_passes=False),
    )
    return k(ev, dst)


def kernel(x, edge_index, edge_attr, Wp, bp,
           Wl1, bl1, Wr1, br1, We1, att1, bo1,
           Wl2, bl2, Wr2, br2, We2, att2, bo2,
           W1, b1, W2, b2, W3, b3):
    src = edge_index[0].astype(jnp.int32)
    dst = edge_index[1].astype(jnp.int32)
    eap = jnp.pad(edge_attr, ((0, 0), (0, 16 - edge_attr.shape[1])))

    S = np.zeros((16, _HID), np.float32)
    for h in range(_H):
        S[h, h * 16:(h + 1) * 16] = 1.0
    S = jnp.asarray(S)

    xl1, xr1 = _tc_proj(x, Wp, bp, Wl1, bl1, Wr1, br1)
    sd1, ev1 = _sc_msg_pass(xl1, xr1, src, dst, eap, We1, att1)
    dd1 = _sc_den_pass(ev1, dst)
    xl2, xr2 = _tc_combine(sd1, dd1, S, bo1, Wl2, bl2, Wr2, br2)
    sd2, ev2 = _sc_msg_pass(xl2, xr2, src, dst, eap, We2, att2)
    dd2 = _sc_den_pass(ev2, dst)
    return _tc_final(sd2, dd2, S, bo2, W1, b1, W2, b2, W3, b3)
